# Initial kernel scaffold; baseline (speedup 1.0000x reference)
import jax
import jax.numpy as jnp
from jax.experimental import pallas as pl


def kernel(input, index1, index2, value):
    raise NotImplementedError("write your pallas kernel here")


# speedup vs baseline: 67.6642x; 67.6642x over previous
import functools

import jax
import jax.numpy as jnp
from jax import lax
from jax.experimental import pallas as pl
from jax.experimental.pallas import tpu as pltpu
from jax.experimental.pallas import tpu_sc as plsc

B = 32
N = 1024
M = 1024
Q = 512
NC = 2
NS = 16
RPT = N // NS
CPT = Q // NS
RPB = 16
IMGS = B // NC
L = 16


def _splat_lane(vec, k):
    dnums = lax.GatherDimensionNumbers(
        offset_dims=(), collapsed_slice_dims=(0,), start_index_map=(0,))
    idx = jnp.full((L, 1), k, jnp.int32)
    return lax.gather(vec, idx, dnums, (1,),
                      mode=lax.GatherScatterMode.PROMISE_IN_BOUNDS)


def _body(inp_hbm, val_hbm, i1_hbm, i2_hbm, out_hbm,
          t_sp, val_col, t_local, out_buf, ts_buf, i2_buf, i1_vmem):
    cid = lax.axis_index("c")
    sid = lax.axis_index("s")
    col0 = sid * CPT
    row0 = sid * RPT

    pltpu.sync_copy(i2_hbm, i2_buf)
    pltpu.sync_copy(i1_hbm, i1_vmem)

    cols_a = lax.iota(jnp.int32, L)
    cols_b = cols_a + L

    def per_image(i, carry):
        img = cid * IMGS + i

        pltpu.sync_copy(val_hbm.at[img, pl.ds(row0, RPT), :],
                        t_sp.at[pl.ds(row0, RPT), :])

        def zero_blk(j, c2):
            for k in range(4):
                t_local[j * 4 + k, pl.ds(0, L)] = jnp.zeros((L,), jnp.float32)
                t_local[j * 4 + k, pl.ds(L, L)] = jnp.zeros((L,), jnp.float32)
            return c2
        lax.fori_loop(0, N // 4, zero_blk, 0)

        plsc.subcore_barrier()

        pltpu.sync_copy(t_sp.at[:, pl.ds(col0, CPT)], val_col)

        def route(pc, c2):
            i1c = i1_vmem[pl.ds(pc * L, L)]
            for k in range(L):
                p = pc * L + k
                rows = _splat_lane(i1c, k)
                plsc.addupdate_scatter(
                    t_local, [rows, cols_a], val_col[p, pl.ds(0, L)])
                plsc.addupdate_scatter(
                    t_local, [rows, cols_b], val_col[p, pl.ds(L, L)])
            return c2
        lax.fori_loop(0, N // L, route, 0)

        pltpu.sync_copy(t_local, t_sp.at[:, pl.ds(col0, CPT)])
        plsc.subcore_barrier()

        for h in range(RPT // RPB):
            r0 = row0 + h * RPB
            pltpu.sync_copy(t_sp.at[pl.ds(r0, RPB), :], ts_buf)
            if h == RPT // RPB - 1:
                plsc.subcore_barrier()
            pltpu.sync_copy(inp_hbm.at[img, pl.ds(r0, RPB), :], out_buf)

            def per_chunk(ch, c2):
                idx = i2_buf[pl.ds(ch * L, L)]
                for row in range(RPB):
                    v = ts_buf[row, pl.ds(ch * L, L)]
                    rows = jnp.full((L,), row, jnp.int32)
                    plsc.addupdate_scatter(out_buf, [rows, idx], v)
                return c2
            lax.fori_loop(0, Q // L, per_chunk, 0)

            pltpu.sync_copy(out_buf, out_hbm.at[img, pl.ds(r0, RPB), :])
        return carry

    lax.fori_loop(0, IMGS, per_image, 0)


@jax.jit
def _scatter_add(inp, val, i1, i2):
    mesh = plsc.VectorSubcoreMesh(core_axis_name="c", subcore_axis_name="s")
    return pl.kernel(
        _body,
        out_type=jax.ShapeDtypeStruct((B, N, M), jnp.float32),
        mesh=mesh,
        compiler_params=pltpu.CompilerParams(
            needs_layout_passes=False, use_tc_tiling_on_sc=False),
        scratch_types=[
            pltpu.VMEM_SHARED((N, Q), jnp.float32),
            pltpu.VMEM((N, CPT), jnp.float32),
            pltpu.VMEM((N, CPT), jnp.float32),
            pltpu.VMEM((RPB, M), jnp.float32),
            pltpu.VMEM((RPB, Q), jnp.float32),
            pltpu.VMEM((Q,), jnp.int32),
            pltpu.VMEM((N,), jnp.int32),
        ],
    )(inp, val, i1, i2)


def kernel(input, index1, index2, value):
    inp = input.reshape(B, N, M)
    val = value.reshape(B, N, Q)
    i1 = index1.reshape(N).astype(jnp.int32)
    i2 = index2.astype(jnp.int32)
    out = _scatter_add(inp, val, i1, i2)
    return out.reshape(input.shape)


# speedup vs baseline: 70.6841x; 1.0446x over previous
import functools

import jax
import jax.numpy as jnp
from jax import lax
from jax.experimental import pallas as pl
from jax.experimental.pallas import tpu as pltpu
from jax.experimental.pallas import tpu_sc as plsc

B = 32
N = 1024
M = 1024
Q = 512
NC = 2
NS = 16
RPT = N // NS
CPT = Q // NS
RPB = 16
NB = RPT // RPB
IMGS = B // NC
L = 16


def _splat_lane(vec, k):
    dnums = lax.GatherDimensionNumbers(
        offset_dims=(), collapsed_slice_dims=(0,), start_index_map=(0,))
    idx = jnp.full((L, 1), k, jnp.int32)
    return lax.gather(vec, idx, dnums, (1,),
                      mode=lax.GatherScatterMode.PROMISE_IN_BOUNDS)


def _body(inp_hbm, val_hbm, i1_hbm, i2_hbm, out_hbm,
          t_sp, val_col, t_local, out_buf, ts_buf, i2_buf, i1_vmem):
    cid = lax.axis_index("c")
    sid = lax.axis_index("s")
    col0 = sid * CPT
    row0 = sid * RPT

    pltpu.sync_copy(i2_hbm, i2_buf)
    pltpu.sync_copy(i1_hbm, i1_vmem)

    cols_a = lax.iota(jnp.int32, L)
    cols_b = cols_a + L

    def per_image(i, carry):
        img = cid * IMGS + i

        def zero_blk(j, c2):
            for k in range(4):
                t_local[j * 4 + k, pl.ds(0, L)] = jnp.zeros((L,), jnp.float32)
                t_local[j * 4 + k, pl.ds(L, L)] = jnp.zeros((L,), jnp.float32)
            return c2
        lax.fori_loop(0, N // 4, zero_blk, 0)

        pltpu.sync_copy(val_hbm.at[img, :, pl.ds(col0, CPT)], val_col)

        def route(pc, c2):
            i1c = i1_vmem[pl.ds(pc * L, L)]
            for k in range(L):
                p = pc * L + k
                rows = _splat_lane(i1c, k)
                plsc.addupdate_scatter(
                    t_local, [rows, cols_a], val_col[p, pl.ds(0, L)])
                plsc.addupdate_scatter(
                    t_local, [rows, cols_b], val_col[p, pl.ds(L, L)])
            return c2
        lax.fori_loop(0, N // L, route, 0)

        pltpu.sync_copy(t_local, t_sp.at[:, pl.ds(col0, CPT)])
        plsc.subcore_barrier()

        for h in range(NB):
            r0 = row0 + h * RPB
            pltpu.sync_copy(t_sp.at[pl.ds(r0, RPB), :], ts_buf)
            if h == NB - 1:
                plsc.subcore_barrier()
            pltpu.sync_copy(inp_hbm.at[img, pl.ds(r0, RPB), :], out_buf)

            def per_chunk(ch, c2):
                idx = i2_buf[pl.ds(ch * L, L)]
                for row in range(RPB):
                    v = ts_buf[row, pl.ds(ch * L, L)]
                    rows = jnp.full((L,), row, jnp.int32)
                    plsc.addupdate_scatter(out_buf, [rows, idx], v)
                return c2
            lax.fori_loop(0, Q // L, per_chunk, 0)

            pltpu.sync_copy(out_buf, out_hbm.at[img, pl.ds(r0, RPB), :])
        return carry

    lax.fori_loop(0, IMGS, per_image, 0)


@jax.jit
def _scatter_add(inp, val, i1, i2):
    mesh = plsc.VectorSubcoreMesh(core_axis_name="c", subcore_axis_name="s")
    return pl.kernel(
        _body,
        out_type=jax.ShapeDtypeStruct((B, N, M), jnp.float32),
        mesh=mesh,
        compiler_params=pltpu.CompilerParams(
            needs_layout_passes=False, use_tc_tiling_on_sc=False),
        scratch_types=[
            pltpu.VMEM_SHARED((N, Q), jnp.float32),
            pltpu.VMEM((N, CPT), jnp.float32),
            pltpu.VMEM((N, CPT), jnp.float32),
            pltpu.VMEM((RPB, M), jnp.float32),
            pltpu.VMEM((RPB, Q), jnp.float32),
            pltpu.VMEM((Q,), jnp.int32),
            pltpu.VMEM((N,), jnp.int32),
        ],
    )(inp, val, i1, i2)


def kernel(input, index1, index2, value):
    inp = input.reshape(B, N, M)
    val = value.reshape(B, N, Q)
    i1 = index1.reshape(N).astype(jnp.int32)
    i2 = index2.astype(jnp.int32)
    out = _scatter_add(inp, val, i1, i2)
    return out.reshape(input.shape)
